# baseline (speedup 1.0000x reference)
import functools

import jax
import jax.numpy as jnp
from jax.experimental import pallas as pl

N = 10000
D = 256
BM = 400


def _xw_kernel(x_ref, w_ref, o_ref):
    o_ref[...] = jnp.dot(
        x_ref[...], w_ref[...], preferred_element_type=jnp.float32
    )


def _layer1_kernel(s_ref, t0_ref, b0_ref, w1_ref, wpt_ref, t1_ref, p_ref):
    h_pre = jnp.dot(s_ref[...], t0_ref[...], preferred_element_type=jnp.float32,
                    precision=jax.lax.Precision.DEFAULT)
    h1 = jax.nn.relu(h_pre + b0_ref[...])
    t1_ref[...] = jnp.dot(
        h1, w1_ref[...], preferred_element_type=jnp.float32
    )
    p_ref[...] = jnp.dot(h1, wpt_ref[...], preferred_element_type=jnp.float32)


def _layer2_kernel(s_ref, t1_ref, b1_ref, wpb_ref, p_ref, bp_ref, o_ref):
    h_pre = jnp.dot(s_ref[...], t1_ref[...], preferred_element_type=jnp.float32,
                    precision=jax.lax.Precision.DEFAULT)
    h2 = jax.nn.relu(h_pre + b1_ref[...])
    o_ref[...] = (
        jnp.dot(h2, wpb_ref[...], preferred_element_type=jnp.float32)
        + p_ref[...]
        + bp_ref[...]
    )


@jax.jit
def kernel(x, support, W0, b0, W1, b1, Wp, bp):
    n_blocks = N // BM
    b0 = b0.reshape(1, D)
    b1 = b1.reshape(1, D)
    bp = bp.reshape(1, D)
    Wp_top = Wp[:D]
    Wp_bot = Wp[D:]

    t0 = pl.pallas_call(
        _xw_kernel,
        grid=(n_blocks,),
        in_specs=[
            pl.BlockSpec((BM, D), lambda i: (i, 0)),
            pl.BlockSpec((D, D), lambda i: (0, 0)),
        ],
        out_specs=pl.BlockSpec((BM, D), lambda i: (i, 0)),
        out_shape=jax.ShapeDtypeStruct((N, D), jnp.float32),
    )(x, W0)

    t1, p = pl.pallas_call(
        _layer1_kernel,
        grid=(n_blocks,),
        in_specs=[
            pl.BlockSpec((BM, N), lambda i: (i, 0)),
            pl.BlockSpec((N, D), lambda i: (0, 0)),
            pl.BlockSpec((1, D), lambda i: (0, 0)),
            pl.BlockSpec((D, D), lambda i: (0, 0)),
            pl.BlockSpec((D, D), lambda i: (0, 0)),
        ],
        out_specs=[
            pl.BlockSpec((BM, D), lambda i: (i, 0)),
            pl.BlockSpec((BM, D), lambda i: (i, 0)),
        ],
        out_shape=[
            jax.ShapeDtypeStruct((N, D), jnp.float32),
            jax.ShapeDtypeStruct((N, D), jnp.float32),
        ],
    )(support, t0, b0, W1, Wp_top)

    out = pl.pallas_call(
        _layer2_kernel,
        grid=(n_blocks,),
        in_specs=[
            pl.BlockSpec((BM, N), lambda i: (i, 0)),
            pl.BlockSpec((N, D), lambda i: (0, 0)),
            pl.BlockSpec((1, D), lambda i: (0, 0)),
            pl.BlockSpec((D, D), lambda i: (0, 0)),
            pl.BlockSpec((BM, D), lambda i: (i, 0)),
            pl.BlockSpec((1, D), lambda i: (0, 0)),
        ],
        out_specs=pl.BlockSpec((BM, D), lambda i: (i, 0)),
        out_shape=jax.ShapeDtypeStruct((N, D), jnp.float32),
    )(support, t1, b1, Wp_bot, p, bp)

    return out


# speedup vs baseline: 1.0298x; 1.0298x over previous
import functools

import jax
import jax.numpy as jnp
from jax.experimental import pallas as pl

N = 10000
D = 256
BM = 400


def _xw_kernel(x_ref, w_ref, o_ref):
    o_ref[...] = jnp.dot(
        x_ref[...], w_ref[...], preferred_element_type=jnp.float32
    ).astype(jnp.bfloat16)


def _layer1_kernel(s_ref, t0_ref, b0_ref, w1_ref, wpt_ref, t1_ref, p_ref):
    s = s_ref[...].astype(jnp.bfloat16)
    h_pre = jnp.dot(s, t0_ref[...], preferred_element_type=jnp.float32)
    h1 = jax.nn.relu(h_pre + b0_ref[...])
    t1_ref[...] = jnp.dot(
        h1, w1_ref[...], preferred_element_type=jnp.float32
    ).astype(jnp.bfloat16)
    p_ref[...] = jnp.dot(
        h1, wpt_ref[...], preferred_element_type=jnp.float32
    ).astype(jnp.bfloat16)


def _layer2_kernel(s_ref, t1_ref, b1_ref, wpb_ref, p_ref, bp_ref, o_ref):
    s = s_ref[...].astype(jnp.bfloat16)
    h_pre = jnp.dot(s, t1_ref[...], preferred_element_type=jnp.float32)
    h2 = jax.nn.relu(h_pre + b1_ref[...])
    o_ref[...] = (
        jnp.dot(h2, wpb_ref[...], preferred_element_type=jnp.float32)
        + p_ref[...].astype(jnp.float32)
        + bp_ref[...]
    )


@jax.jit
def kernel(x, support, W0, b0, W1, b1, Wp, bp):
    n_blocks = N // BM
    b0 = b0.reshape(1, D)
    b1 = b1.reshape(1, D)
    bp = bp.reshape(1, D)
    Wp_top = Wp[:D]
    Wp_bot = Wp[D:]

    t0 = pl.pallas_call(
        _xw_kernel,
        grid=(n_blocks,),
        in_specs=[
            pl.BlockSpec((BM, D), lambda i: (i, 0)),
            pl.BlockSpec((D, D), lambda i: (0, 0)),
        ],
        out_specs=pl.BlockSpec((BM, D), lambda i: (i, 0)),
        out_shape=jax.ShapeDtypeStruct((N, D), jnp.bfloat16),
    )(x, W0)

    t1, p = pl.pallas_call(
        _layer1_kernel,
        grid=(n_blocks,),
        in_specs=[
            pl.BlockSpec((BM, N), lambda i: (i, 0)),
            pl.BlockSpec((N, D), lambda i: (0, 0)),
            pl.BlockSpec((1, D), lambda i: (0, 0)),
            pl.BlockSpec((D, D), lambda i: (0, 0)),
            pl.BlockSpec((D, D), lambda i: (0, 0)),
        ],
        out_specs=[
            pl.BlockSpec((BM, D), lambda i: (i, 0)),
            pl.BlockSpec((BM, D), lambda i: (i, 0)),
        ],
        out_shape=[
            jax.ShapeDtypeStruct((N, D), jnp.bfloat16),
            jax.ShapeDtypeStruct((N, D), jnp.bfloat16),
        ],
    )(support, t0, b0, W1, Wp_top)

    out = pl.pallas_call(
        _layer2_kernel,
        grid=(n_blocks,),
        in_specs=[
            pl.BlockSpec((BM, N), lambda i: (i, 0)),
            pl.BlockSpec((N, D), lambda i: (0, 0)),
            pl.BlockSpec((1, D), lambda i: (0, 0)),
            pl.BlockSpec((D, D), lambda i: (0, 0)),
            pl.BlockSpec((BM, D), lambda i: (i, 0)),
            pl.BlockSpec((1, D), lambda i: (0, 0)),
        ],
        out_specs=pl.BlockSpec((BM, D), lambda i: (i, 0)),
        out_shape=jax.ShapeDtypeStruct((N, D), jnp.float32),
    )(support, t1, b1, Wp_bot, p, bp)

    return out


# speedup vs baseline: 1.0950x; 1.0633x over previous
import jax
import jax.numpy as jnp
from jax.experimental import pallas as pl
from jax.experimental.pallas import tpu as pltpu

N = 10000
D = 256
BM = 400


def _layer1_kernel(
    s_ref, x_ref, w0_ref, b0_ref, w1_ref, wpt_ref, t1_ref, p_ref, t0_ref
):
    @pl.when(pl.program_id(0) == 0)
    def _():
        xb = x_ref[...].astype(jnp.bfloat16)
        w0 = w0_ref[...].astype(jnp.bfloat16)
        t0_ref[...] = jnp.dot(
            xb, w0, preferred_element_type=jnp.float32
        ).astype(jnp.bfloat16)

    s = s_ref[...].astype(jnp.bfloat16)
    h_pre = jnp.dot(s, t0_ref[...], preferred_element_type=jnp.float32)
    h1 = jax.nn.relu(h_pre + b0_ref[...])
    t1_ref[...] = jnp.dot(
        h1, w1_ref[...], preferred_element_type=jnp.float32
    ).astype(jnp.bfloat16)
    p_ref[...] = jnp.dot(
        h1, wpt_ref[...], preferred_element_type=jnp.float32
    ).astype(jnp.bfloat16)


def _layer2_kernel(s_ref, t1_ref, b1_ref, wpb_ref, p_ref, bp_ref, o_ref):
    s = s_ref[...].astype(jnp.bfloat16)
    h_pre = jnp.dot(s, t1_ref[...], preferred_element_type=jnp.float32)
    h2 = jax.nn.relu(h_pre + b1_ref[...])
    o_ref[...] = (
        jnp.dot(h2, wpb_ref[...], preferred_element_type=jnp.float32)
        + p_ref[...].astype(jnp.float32)
        + bp_ref[...]
    )


@jax.jit
def kernel(x, support, W0, b0, W1, b1, Wp, bp):
    n_blocks = N // BM
    b0 = b0.reshape(1, D)
    b1 = b1.reshape(1, D)
    bp = bp.reshape(1, D)
    Wp_top = Wp[:D]
    Wp_bot = Wp[D:]

    t1, p = pl.pallas_call(
        _layer1_kernel,
        grid=(n_blocks,),
        in_specs=[
            pl.BlockSpec((BM, N), lambda i: (i, 0)),
            pl.BlockSpec((N, D), lambda i: (0, 0)),
            pl.BlockSpec((D, D), lambda i: (0, 0)),
            pl.BlockSpec((1, D), lambda i: (0, 0)),
            pl.BlockSpec((D, D), lambda i: (0, 0)),
            pl.BlockSpec((D, D), lambda i: (0, 0)),
        ],
        out_specs=[
            pl.BlockSpec((BM, D), lambda i: (i, 0)),
            pl.BlockSpec((BM, D), lambda i: (i, 0)),
        ],
        out_shape=[
            jax.ShapeDtypeStruct((N, D), jnp.bfloat16),
            jax.ShapeDtypeStruct((N, D), jnp.bfloat16),
        ],
        scratch_shapes=[pltpu.VMEM((N, D), jnp.bfloat16)],
    )(support, x, W0, b0, W1, Wp_top)

    out = pl.pallas_call(
        _layer2_kernel,
        grid=(n_blocks,),
        in_specs=[
            pl.BlockSpec((BM, N), lambda i: (i, 0)),
            pl.BlockSpec((N, D), lambda i: (0, 0)),
            pl.BlockSpec((1, D), lambda i: (0, 0)),
            pl.BlockSpec((D, D), lambda i: (0, 0)),
            pl.BlockSpec((BM, D), lambda i: (i, 0)),
            pl.BlockSpec((1, D), lambda i: (0, 0)),
        ],
        out_specs=pl.BlockSpec((BM, D), lambda i: (i, 0)),
        out_shape=jax.ShapeDtypeStruct((N, D), jnp.float32),
    )(support, t1, b1, Wp_bot, p, bp)

    return out


# speedup vs baseline: 1.1194x; 1.0223x over previous
import jax
import jax.numpy as jnp
from jax.experimental import pallas as pl
from jax.experimental.pallas import tpu as pltpu

N = 10000
D = 256
BM = 400


def _gcn_kernel(
    s_ref, x_ref, w0_ref, b0_ref, w1_ref, b1_ref, wp_ref, bp_ref,
    o_ref, t0_ref, t1_ref, p_ref,
):
    l = pl.program_id(0)
    i = pl.program_id(1)
    s = s_ref[...].astype(jnp.bfloat16)

    @pl.when(jnp.logical_and(l == 0, i == 0))
    def _():
        w0 = w0_ref[...].astype(jnp.bfloat16)
        t0_ref[...] = jnp.dot(
            x_ref[...], w0, preferred_element_type=jnp.float32
        ).astype(jnp.bfloat16)

    @pl.when(l == 0)
    def _():
        h_pre = jnp.dot(s, t0_ref[...], preferred_element_type=jnp.float32)
        h1 = jax.nn.relu(h_pre + b0_ref[...])
        t1_ref[pl.ds(i * BM, BM), :] = jnp.dot(
            h1, w1_ref[...], preferred_element_type=jnp.float32
        ).astype(jnp.bfloat16)
        p_ref[pl.ds(i * BM, BM), :] = jnp.dot(
            h1, wp_ref[:D], preferred_element_type=jnp.float32
        ).astype(jnp.bfloat16)

    @pl.when(l == 1)
    def _():
        h_pre = jnp.dot(s, t1_ref[...], preferred_element_type=jnp.float32)
        h2 = jax.nn.relu(h_pre + b1_ref[...])
        o_ref[...] = (
            jnp.dot(h2, wp_ref[D:], preferred_element_type=jnp.float32)
            + p_ref[pl.ds(i * BM, BM), :].astype(jnp.float32)
            + bp_ref[...]
        )


@jax.jit
def kernel(x, support, W0, b0, W1, b1, Wp, bp):
    n_blocks = N // BM
    b0 = b0.reshape(1, D)
    b1 = b1.reshape(1, D)
    bp = bp.reshape(1, D)

    out = pl.pallas_call(
        _gcn_kernel,
        grid=(2, n_blocks),
        in_specs=[
            pl.BlockSpec((BM, N), lambda l, i: (i, 0)),
            pl.BlockSpec((N, D), lambda l, i: (0, 0)),
            pl.BlockSpec((D, D), lambda l, i: (0, 0)),
            pl.BlockSpec((1, D), lambda l, i: (0, 0)),
            pl.BlockSpec((D, D), lambda l, i: (0, 0)),
            pl.BlockSpec((1, D), lambda l, i: (0, 0)),
            pl.BlockSpec((2 * D, D), lambda l, i: (0, 0)),
            pl.BlockSpec((1, D), lambda l, i: (0, 0)),
        ],
        out_specs=pl.BlockSpec((BM, D), lambda l, i: (l * i, 0)),
        out_shape=jax.ShapeDtypeStruct((N, D), jnp.float32),
        scratch_shapes=[
            pltpu.VMEM((N, D), jnp.bfloat16),
            pltpu.VMEM((N, D), jnp.bfloat16),
            pltpu.VMEM((N, D), jnp.bfloat16),
        ],
        compiler_params=pltpu.CompilerParams(
            vmem_limit_bytes=int(63.5 * 1024 * 1024)
        ),
    )(support, x.astype(jnp.bfloat16), W0, b0, W1, b1, Wp, bp)

    return out
